# baseline (speedup 1.0000x reference)
import jax
import jax.numpy as jnp
from jax import lax
from jax.experimental import pallas as pl
from jax.experimental.pallas import tpu as pltpu
from jax.experimental.pallas import tpu_sc as plsc

BATCH = 16384
NOBJ = 50
DIM = 16
NWORD = DIM // 2
SIZE = 1000000
NC = 2
NS = 16
NW = NC * NS
ROWS_PER_W = BATCH // NW
CHUNK = 64
NCHUNK = ROWS_PER_W // CHUNK
IDX_PER_CHUNK = CHUNK * NOBJ
OUT_PER_CHUNK = CHUNK * (NOBJ - 1)

_HI = jnp.int32(-65536)


def _unpack(v):
    hi = plsc.bitcast(v & _HI, jnp.float32)
    lo = plsc.bitcast(v << 16, jnp.float32)
    return hi, lo


def _sc_body(idx_hbm, lt_hbm, out_hbm, idx_v, rows_v, out_v, gsem):
    wid = lax.axis_index("s") * NC + lax.axis_index("c")
    iota = lax.iota(jnp.int32, 16)

    for c in range(NCHUNK):
        row_base = wid * ROWS_PER_W + c * CHUNK
        pltpu.sync_copy(idx_hbm.at[pl.ds(row_base, CHUNK), :], idx_v)
        copies = [
            pltpu.async_copy(
                lt_hbm.at[idx_v.at[i]],
                rows_v.at[pl.ds(i * NOBJ, NOBJ)],
                gsem,
            )
            for i in range(CHUNK)
        ]
        for cp in copies:
            cp.wait()

        for g in range(4):
            row0 = (g * 16 + iota) * NOBJ
            outb = (g * 16 + iota) * (NOBJ - 1)
            svec = []
            for k in range(NWORD):
                w = plsc.load_gather(rows_v, [row0, (iota + k) & 7])
                svec.append(_unpack(w))

            @pl.loop(0, NOBJ - 1)
            def _(j, row0=row0, outb=outb, svec=svec):
                orow = row0 + (j + 1)
                acc = None
                for k in range(NWORD):
                    w = plsc.load_gather(rows_v, [orow, (iota + k) & 7])
                    ohi, olo = _unpack(w)
                    shi, slo = svec[k]
                    th = shi - ohi
                    tl = slo - olo
                    part = th * th + tl * tl
                    acc = part if acc is None else acc + part
                plsc.store_scatter(out_v, [outb + j], acc)

        pltpu.sync_copy(
            out_v,
            out_hbm.at[pl.ds(wid * ROWS_PER_W * (NOBJ - 1) + c * OUT_PER_CHUNK,
                             OUT_PER_CHUNK)],
        )


@jax.jit
def _run(idx, lti):
    mesh = plsc.VectorSubcoreMesh(core_axis_name="c", subcore_axis_name="s")
    flat = pl.kernel(
        _sc_body,
        out_type=jax.ShapeDtypeStruct((BATCH * (NOBJ - 1),), jnp.float32),
        mesh=mesh,
        scratch_types=[
            pltpu.VMEM((CHUNK, NOBJ), jnp.int32),
            pltpu.VMEM((IDX_PER_CHUNK, NWORD), jnp.int32),
            pltpu.VMEM((OUT_PER_CHUNK,), jnp.float32),
            pltpu.SemaphoreType.DMA,
        ],
        compiler_params=pltpu.CompilerParams(
            needs_layout_passes=False,
            use_tc_tiling_on_sc=False,
        ),
    )(idx, lti)
    return flat.reshape(BATCH, NOBJ - 1)


def kernel(inputs, lt):
    lti = lax.bitcast_convert_type(
        lt.astype(jnp.bfloat16).reshape(SIZE, NWORD, 2), jnp.int32
    )
    return _run(inputs.astype(jnp.int32), lti)


# speedup vs baseline: 1.9572x; 1.9572x over previous
import jax
import jax.numpy as jnp
from jax import lax
from jax.experimental import pallas as pl
from jax.experimental.pallas import tpu as pltpu
from jax.experimental.pallas import tpu_sc as plsc

BATCH = 16384
NOBJ = 50
DIM = 16
NC = 2
NS = 16
NW = NC * NS
ROWS_PER_W = BATCH // NW
CHUNK = 64
NCHUNK = ROWS_PER_W // CHUNK
IDX_PER_CHUNK = CHUNK * NOBJ
OUT_PER_CHUNK = CHUNK * (NOBJ - 1)


def _sc_body(idx_hbm, lt_hbm, out_hbm, idx_v, rows_v, out_v, gsem):
    wid = lax.axis_index("s") * NC + lax.axis_index("c")
    iota = lax.iota(jnp.int32, 16)

    for c in range(NCHUNK):
        row_base = wid * ROWS_PER_W + c * CHUNK
        pltpu.sync_copy(idx_hbm.at[pl.ds(row_base, CHUNK), :], idx_v)
        copies = [
            pltpu.async_copy(
                lt_hbm.at[idx_v.at[i]],
                rows_v.at[pl.ds(i * NOBJ, NOBJ)],
                gsem,
            )
            for i in range(CHUNK)
        ]
        for cp in copies:
            cp.wait()

        for g in range(4):
            row0 = (g * 16 + iota) * NOBJ
            outb = (g * 16 + iota) * (NOBJ - 1)
            svec = [
                plsc.load_gather(rows_v, [row0, (iota + k) & 15])
                for k in range(DIM)
            ]

            @pl.loop(0, NOBJ - 1)
            def _(j, row0=row0, outb=outb, svec=svec):
                orow = row0 + (j + 1)
                acc = None
                for k in range(DIM):
                    o = plsc.load_gather(rows_v, [orow, (iota + k) & 15])
                    t = svec[k] - o
                    acc = t * t if acc is None else acc + t * t
                plsc.store_scatter(out_v, [outb + j], acc)

        pltpu.sync_copy(
            out_v,
            out_hbm.at[pl.ds(wid * ROWS_PER_W * (NOBJ - 1) + c * OUT_PER_CHUNK,
                             OUT_PER_CHUNK)],
        )


@jax.jit
def _run(idx, lt):
    mesh = plsc.VectorSubcoreMesh(core_axis_name="c", subcore_axis_name="s")
    flat = pl.kernel(
        _sc_body,
        out_type=jax.ShapeDtypeStruct((BATCH * (NOBJ - 1),), jnp.float32),
        mesh=mesh,
        scratch_types=[
            pltpu.VMEM((CHUNK, NOBJ), jnp.int32),
            pltpu.VMEM((IDX_PER_CHUNK, DIM), jnp.float32),
            pltpu.VMEM((OUT_PER_CHUNK,), jnp.float32),
            pltpu.SemaphoreType.DMA,
        ],
        compiler_params=pltpu.CompilerParams(
            needs_layout_passes=False,
            use_tc_tiling_on_sc=False,
        ),
    )(idx, lt)
    return flat.reshape(BATCH, NOBJ - 1)


def kernel(inputs, lt):
    return _run(inputs.astype(jnp.int32), lt)
